# baseline (speedup 1.0000x reference)
import functools

import jax
import jax.numpy as jnp
from jax import lax
from jax.experimental import pallas as pl
from jax.experimental.pallas import tpu as pltpu
from jax.experimental.pallas import tpu_sc as plsc

NUM_CORES = 2
NUM_SUBCORES = 16
LANES = 16
NUM_WORKERS = NUM_CORES * NUM_SUBCORES

BATCH = 16384
EMBED = 16
BPW = BATCH // NUM_WORKERS
CHUNK = 128
NCHUNKS = BPW // CHUNK


def _sc_body(x_hbm, table_hbm, params_hbm, out_hbm, idx_v, rows_v, out_v,
             par_v, sem):
    wid = lax.axis_index("s") * NUM_CORES + lax.axis_index("c")
    base = wid * BPW

    pltpu.sync_copy(params_hbm, par_v)
    pltpu.sync_copy(x_hbm.at[wid], idx_v)

    copies = [
        pltpu.async_copy(
            table_hbm.at[idx_v.at[j]],
            rows_v.at[pl.ds(j * CHUNK, CHUNK)],
            sem,
        )
        for j in range(NCHUNKS)
    ]
    for c in copies:
        c.wait()

    lane = lax.iota(jnp.int32, LANES)
    wvec = par_v[pl.ds(0, LANES)]
    bvec = par_v[pl.ds(LANES, LANES)]
    bias = bvec[0]

    def block(t, carry):
        acc = jnp.zeros((LANES,), jnp.float32)
        for j in range(LANES):
            r = rows_v[t * LANES + j]
            s = jnp.sum(r * wvec)
            acc = jnp.where(lane == j, s, acc)
        out_v[pl.ds(t * LANES, LANES)] = acc + bias
        return carry

    lax.fori_loop(0, BPW // LANES, block, 0)

    pltpu.sync_copy(out_v, out_hbm.at[pl.ds(base, BPW)])


@functools.partial(
    pl.kernel,
    out_type=jax.ShapeDtypeStruct((BATCH,), jnp.float32),
    mesh=plsc.VectorSubcoreMesh(core_axis_name="c", subcore_axis_name="s"),
    scratch_types=[
        pltpu.VMEM((NCHUNKS, CHUNK), jnp.int32),
        pltpu.VMEM((BPW, EMBED), jnp.float32),
        pltpu.VMEM((BPW,), jnp.float32),
        pltpu.VMEM((32,), jnp.float32),
        pltpu.SemaphoreType.DMA,
    ],
    compiler_params=pltpu.CompilerParams(
        needs_layout_passes=False, use_tc_tiling_on_sc=False
    ),
)
def _sc_kernel(x_hbm, table_hbm, params_hbm, out_hbm, idx_v, rows_v, out_v,
               par_v, sem):
    _sc_body(x_hbm, table_hbm, params_hbm, out_hbm, idx_v, rows_v, out_v,
             par_v, sem)


def kernel(x, table, fc_w, fc_b):
    xi = x.astype(jnp.int32).reshape(NUM_WORKERS, NCHUNKS, CHUNK)
    params = jnp.concatenate(
        [fc_w.reshape(-1).astype(jnp.float32), fc_b.astype(jnp.float32)]
    )
    params = jnp.pad(params, (0, 32 - params.shape[0]))
    out = _sc_kernel(xi, table.astype(jnp.float32), params)
    return out.reshape(x.shape[0], 1)
